# Initial kernel scaffold; baseline (speedup 1.0000x reference)
import jax
import jax.numpy as jnp
from jax.experimental import pallas as pl


def kernel(x, edge_index, W_proj, b_proj, W0, b0, g0, be0, W1, b1, g1, be1):
    raise NotImplementedError("write your pallas kernel here")


# speedup vs baseline: 18.0483x; 18.0483x over previous
import functools

import jax
import jax.numpy as jnp
from jax import lax
from jax.experimental import pallas as pl
from jax.experimental.pallas import tpu as pltpu
from jax.experimental.pallas import tpu_sc as plsc

_F32 = jnp.float32

_K = 80
_NPAD = 10240
_NPASS = 4
_SEG = _NPAD // _NPASS
_ACC = _SEG + 128
_R = 2000


def _tc_proj(x, WpT, bp, W0T, b0):
    B, N, C = x.shape
    grid = (B, N // _R)

    def body(x_ref, wp_ref, bp_ref, w0_ref, b0_ref, h_ref, t_ref):
        h = jnp.dot(x_ref[0], wp_ref[...], preferred_element_type=_F32)
        h = h + bp_ref[...]
        h_ref[0] = h
        t_ref[0] = jnp.dot(h, w0_ref[...], preferred_element_type=_F32) + b0_ref[...]

    blk = pl.BlockSpec((1, _R, C), lambda b, i: (b, i, 0))
    blk_w = pl.BlockSpec((C, C), lambda b, i: (0, 0))
    blk_b = pl.BlockSpec((1, C), lambda b, i: (0, 0))
    return pl.pallas_call(
        body,
        grid=grid,
        in_specs=[blk, blk_w, blk_b, blk_w, blk_b],
        out_specs=[blk, blk],
        out_shape=[
            jax.ShapeDtypeStruct((B, N, C), _F32),
            jax.ShapeDtypeStruct((B, _NPAD, C), _F32),
        ],
    )(x, WpT, bp.reshape(1, C), W0T, b0.reshape(1, C))


def _tc_post(sums, cnt, g, be, n_nodes, WT=None, b=None, shortcut=None):
    B, NP, C = sums.shape
    N = n_nodes
    grid = (B, N // _R)

    blk = pl.BlockSpec((1, _R, C), lambda b_, i: (b_, i, 0))
    blk_cnt = pl.BlockSpec((_R, C), lambda b_, i: (i, 0))
    blk_vec = pl.BlockSpec((1, C), lambda b_, i: (0, 0))
    blk_w = pl.BlockSpec((C, C), lambda b_, i: (0, 0))

    def stage(s_ref, c_ref, g_ref, be_ref):
        cnt_col = c_ref[:, :1]
        mean = s_ref[0] / jnp.maximum(cnt_col, 1.0)
        mu = jnp.mean(mean, axis=-1, keepdims=True)
        var = jnp.mean((mean - mu) ** 2, axis=-1, keepdims=True)
        h = (mean - mu) * lax.rsqrt(var + 1e-5) * g_ref[...] + be_ref[...]
        return h * jax.nn.sigmoid(h)

    if WT is not None:
        def body(s_ref, c_ref, g_ref, be_ref, w_ref, b_ref, o_ref):
            h = stage(s_ref, c_ref, g_ref, be_ref)
            o_ref[0] = jnp.dot(h, w_ref[...], preferred_element_type=_F32) + b_ref[...]

        in_specs = [blk, blk_cnt, blk_vec, blk_vec, blk_w, blk_vec]
        args = (sums, cnt, g.reshape(1, C), be.reshape(1, C), WT, b.reshape(1, C))
        out_shape = jax.ShapeDtypeStruct((B, _NPAD, C), _F32)
    else:
        def body(s_ref, c_ref, g_ref, be_ref, sc_ref, o_ref):
            o_ref[0] = stage(s_ref, c_ref, g_ref, be_ref) + sc_ref[0]

        in_specs = [blk, blk_cnt, blk_vec, blk_vec, blk]
        args = (sums, cnt, g.reshape(1, C), be.reshape(1, C), shortcut)
        out_shape = jax.ShapeDtypeStruct((B, N, C), _F32)

    return pl.pallas_call(
        body,
        grid=grid,
        in_specs=in_specs,
        out_specs=blk,
        out_shape=out_shape,
    )(*args)


def _sc_agg(table, src_all, dst4, zeros_acc, ones_rows=None):
    C = table.shape[1]
    cps = dst4.shape[2]
    with_cnt = ones_rows is not None
    info = plsc.get_sparse_core_info()
    NS = info.num_subcores
    zps = _ACC // NS
    wps = _SEG // NS
    mesh = plsc.VectorSubcoreMesh(core_axis_name="c", subcore_axis_name="s")

    out_type = jax.ShapeDtypeStruct((2 * _NPAD, C), _F32)
    scratch = [
        pltpu.VMEM((cps, _K), jnp.int32),
        pltpu.VMEM((cps, _K), jnp.int32),
        pltpu.VMEM((_K, C), _F32),
        pltpu.VMEM_SHARED((_ACC, C), _F32),
        pltpu.SemaphoreType.DMA,
    ]
    if with_cnt:
        out_type = [out_type, jax.ShapeDtypeStruct((_NPAD, C), _F32)]
        scratch.insert(3, pltpu.VMEM((_K, C), _F32))

    @functools.partial(pl.kernel, out_type=out_type, mesh=mesh,
                       scratch_types=scratch)
    def agg(*refs):
        if with_cnt:
            (table_h, src_h, dst_h, zacc_h, ones_h, out_h, cnt_h,
             src_v, d_v, ones_v, rows_v, acc_sh, sem) = refs
        else:
            (table_h, src_h, dst_h, zacc_h, out_h,
             src_v, d_v, rows_v, acc_sh, sem) = refs
        c = lax.axis_index("c")
        s = lax.axis_index("s")
        z0 = s * zps
        w0 = s * wps

        def zero_acc():
            pltpu.sync_copy(zacc_h.at[pl.ds(z0, zps)],
                            acc_sh.at[pl.ds(z0, zps)])

        zero_acc()
        pltpu.sync_copy(src_h.at[c, s], src_v)
        if with_cnt:
            pltpu.sync_copy(ones_h, ones_v)
        plsc.subcore_barrier()

        for p in range(_NPASS):
            pltpu.sync_copy(dst_h.at[p, s], d_v)

            def chunk_body(j, carry):
                pltpu.async_copy(table_h.at[src_v.at[j]], rows_v, sem).wait()
                pltpu.sync_copy(rows_v, acc_sh.at[d_v.at[j]], add=True)
                return carry

            lax.fori_loop(0, cps, chunk_body, 0)
            plsc.subcore_barrier()
            pltpu.sync_copy(
                acc_sh.at[pl.ds(w0, wps)],
                out_h.at[pl.ds(c * _NPAD + p * _SEG + w0, wps)])
            if p + 1 < _NPASS or with_cnt:
                zero_acc()
                plsc.subcore_barrier()

        if with_cnt:
            for k in range(2):
                q = 2 * c + k
                pltpu.sync_copy(dst_h.at[q, s], d_v)

                def cnt_body(j, carry):
                    pltpu.sync_copy(ones_v, acc_sh.at[d_v.at[j]], add=True)
                    return carry

                lax.fori_loop(0, cps, cnt_body, 0)
                plsc.subcore_barrier()
                pltpu.sync_copy(
                    acc_sh.at[pl.ds(w0, wps)],
                    cnt_h.at[pl.ds((2 * c + k) * _SEG + w0, wps)])
                if k == 0:
                    zero_acc()
                    plsc.subcore_barrier()

    return agg(table, src_all, dst4, zeros_acc,
               *((ones_rows,) if with_cnt else ()))


def kernel(x, edge_index, W_proj, b_proj, W0, b0, g0, be0, W1, b1, g1, be1):
    B, N, C = x.shape
    E = edge_index.shape[1]

    h_sc, t0 = _tc_proj(x, W_proj.T, b_proj, W0.T, b0)

    src = edge_index[0].astype(jnp.int32)
    dst = edge_index[1].astype(jnp.int32)
    ns = 16
    cps = E // _K // ns
    src_all = jnp.stack([src, src + _NPAD]).reshape(2, ns, cps, _K)
    dloc = dst[None, :] - (jnp.arange(_NPASS, dtype=jnp.int32) * _SEG)[:, None]
    dst4 = jnp.where((dloc >= 0) & (dloc < _SEG), dloc, _SEG)
    dst4 = dst4.reshape(_NPASS, ns, cps, _K)
    zeros_acc = jnp.zeros((_ACC, C), _F32)
    ones_rows = jnp.ones((_K, C), _F32)

    def flat(t):
        return t.reshape(2 * _NPAD, C)

    def unflat(t):
        return t.reshape(2, _NPAD, C)

    s0, cnt = _sc_agg(flat(t0), src_all, dst4, zeros_acc, ones_rows)
    t1 = _tc_post(unflat(s0), cnt, g0, be0, N, WT=W1.T, b=b1)
    s1 = _sc_agg(flat(t1), src_all, dst4, zeros_acc)
    out = _tc_post(unflat(s1), cnt, g1, be1, N, shortcut=h_sc)
    return out


# speedup vs baseline: 43.1034x; 2.3882x over previous
import functools

import jax
import jax.numpy as jnp
from jax import lax
from jax.experimental import pallas as pl
from jax.experimental.pallas import tpu as pltpu
from jax.experimental.pallas import tpu_sc as plsc

_F32 = jnp.float32

_K = 80
_NPAD = 10240
_NPASS = 8
_SEG = _NPAD // _NPASS
_ACC = _SEG + 128
_R = 2000


def _tc_proj(x, WpT, bp, W0T, b0):
    B, N, C = x.shape
    grid = (B, N // _R)

    def body(x_ref, wp_ref, bp_ref, w0_ref, b0_ref, h_ref, t_ref):
        h = jnp.dot(x_ref[0], wp_ref[...], preferred_element_type=_F32)
        h = h + bp_ref[...]
        h_ref[0] = h
        t_ref[0] = jnp.dot(h, w0_ref[...], preferred_element_type=_F32) + b0_ref[...]

    blk = pl.BlockSpec((1, _R, C), lambda b, i: (b, i, 0))
    blk_w = pl.BlockSpec((C, C), lambda b, i: (0, 0))
    blk_b = pl.BlockSpec((1, C), lambda b, i: (0, 0))
    return pl.pallas_call(
        body,
        grid=grid,
        in_specs=[blk, blk_w, blk_b, blk_w, blk_b],
        out_specs=[blk, blk],
        out_shape=[
            jax.ShapeDtypeStruct((B, N, C), _F32),
            jax.ShapeDtypeStruct((B, _NPAD, C), _F32),
        ],
    )(x, WpT, bp.reshape(1, C), W0T, b0.reshape(1, C))


def _tc_post(sums, cnt, g, be, n_nodes, WT=None, b=None, shortcut=None):
    B, NP, C = sums.shape
    N = n_nodes
    grid = (B, N // _R)

    blk = pl.BlockSpec((1, _R, C), lambda b_, i: (b_, i, 0))
    blk_cnt = pl.BlockSpec((_R, C), lambda b_, i: (i, 0))
    blk_vec = pl.BlockSpec((1, C), lambda b_, i: (0, 0))
    blk_w = pl.BlockSpec((C, C), lambda b_, i: (0, 0))

    def stage(s_ref, c_ref, g_ref, be_ref):
        cnt_col = c_ref[:, :1]
        mean = s_ref[0] / jnp.maximum(cnt_col, 1.0)
        mu = jnp.mean(mean, axis=-1, keepdims=True)
        var = jnp.mean((mean - mu) ** 2, axis=-1, keepdims=True)
        h = (mean - mu) * lax.rsqrt(var + 1e-5) * g_ref[...] + be_ref[...]
        return h * jax.nn.sigmoid(h)

    if WT is not None:
        def body(s_ref, c_ref, g_ref, be_ref, w_ref, b_ref, o_ref):
            h = stage(s_ref, c_ref, g_ref, be_ref)
            o_ref[0] = jnp.dot(h, w_ref[...], preferred_element_type=_F32) + b_ref[...]

        in_specs = [blk, blk_cnt, blk_vec, blk_vec, blk_w, blk_vec]
        args = (sums, cnt, g.reshape(1, C), be.reshape(1, C), WT, b.reshape(1, C))
        out_shape = jax.ShapeDtypeStruct((B, _NPAD, C), _F32)
    else:
        def body(s_ref, c_ref, g_ref, be_ref, sc_ref, o_ref):
            o_ref[0] = stage(s_ref, c_ref, g_ref, be_ref) + sc_ref[0]

        in_specs = [blk, blk_cnt, blk_vec, blk_vec, blk]
        args = (sums, cnt, g.reshape(1, C), be.reshape(1, C), shortcut)
        out_shape = jax.ShapeDtypeStruct((B, N, C), _F32)

    return pl.pallas_call(
        body,
        grid=grid,
        in_specs=in_specs,
        out_specs=blk,
        out_shape=out_shape,
    )(*args)


def _sc_agg(table, src_all, dst3, zeros_acc, ones_rows=None):
    C = table.shape[1]
    cps = dst3.shape[1]
    ne = cps * _K
    with_cnt = ones_rows is not None
    info = plsc.get_sparse_core_info()
    NS = info.num_subcores
    zps = _ACC // NS
    wps = _SEG // NS
    mesh = plsc.VectorSubcoreMesh(core_axis_name="c", subcore_axis_name="s")

    out_type = jax.ShapeDtypeStruct((2 * _NPAD, C), _F32)
    scratch = [
        pltpu.VMEM((cps, _K), jnp.int32),
        pltpu.VMEM((cps, _K), jnp.int32),
        pltpu.VMEM((ne + 2 * _K,), jnp.int32),
        pltpu.VMEM((ne + 2 * _K,), jnp.int32),
        pltpu.VMEM((1, _K), jnp.int32),
        pltpu.VMEM((_K, C), _F32),
        pltpu.VMEM_SHARED((_ACC, C), _F32),
        pltpu.SemaphoreType.DMA,
    ]
    if with_cnt:
        out_type = [out_type, jax.ShapeDtypeStruct((_NPAD, C), _F32)]

    @functools.partial(
        pl.kernel, out_type=out_type, mesh=mesh, scratch_types=scratch,
        compiler_params=pltpu.CompilerParams(needs_layout_passes=False))
    def agg(*refs):
        if with_cnt:
            (table_h, src_h, dst_h, zacc_h, ones_h, out_h, cnt_h,
             src_v, d_v, srcc, dstc, idx2, rows_v, acc_sh, sem) = refs
        else:
            (table_h, src_h, dst_h, zacc_h, out_h,
             src_v, d_v, srcc, dstc, idx2, rows_v, acc_sh, sem) = refs
        c = lax.axis_index("c")
        s = lax.axis_index("s")
        z0 = s * zps
        w0 = s * wps

        def zero_acc():
            pltpu.sync_copy(zacc_h.at[pl.ds(z0, zps)],
                            acc_sh.at[pl.ds(z0, zps)])

        zero_acc()
        pltpu.sync_copy(src_h.at[c, s], src_v)
        pltpu.sync_copy(dst_h.at[s], d_v)
        plsc.subcore_barrier()

        def pass_body(p, carry):
            lo = p * _SEG

            def row_body(t, pos):
                parts = []
                for l in range(_K // 16):
                    d = d_v[t, pl.ds(l * 16, 16)]
                    sv = src_v[t, pl.ds(l * 16, 16)]
                    local = d - lo
                    m = (local >= 0) & (local < _SEG)
                    mi = m.astype(jnp.int32)
                    cs = plsc.cumsum(mi)
                    parts.append((local, sv, m, mi, cs))
                for local, sv, m, mi, cs in parts:
                    tgt = pos + cs - mi
                    plsc.store_scatter(dstc, [tgt], local, mask=m)
                    plsc.store_scatter(srcc, [tgt], sv, mask=m)
                    pos = pos + jnp.sum(mi, axis=0)
                return pos

            pos = lax.fori_loop(0, cps, row_body, 0)
            padd = jnp.full((16,), _SEG, jnp.int32)
            pads = jnp.broadcast_to(c * _NPAD, (16,)).astype(jnp.int32)
            for l in range(6):
                dstc[pl.ds(pos + l * 16, 16)] = padd
                srcc[pl.ds(pos + l * 16, 16)] = pads
            nch = (pos + _K - 1) // _K

            def load_idx2(j):
                for l in range(_K // 16):
                    idx2[0, pl.ds(l * 16, 16)] = dstc[pl.ds(j * _K + l * 16, 16)]

            def chunk_body(j, carry):
                load_idx2(j)
                pltpu.async_copy(table_h.at[srcc.at[pl.ds(j * _K, _K)]],
                                 rows_v, sem).wait()
                pltpu.sync_copy(rows_v, acc_sh.at[idx2.at[0]], add=True)
                return carry

            lax.fori_loop(0, nch, chunk_body, 0)
            plsc.subcore_barrier()
            pltpu.sync_copy(
                acc_sh.at[pl.ds(w0, wps)],
                out_h.at[pl.ds(c * _NPAD + p * _SEG + w0, wps)])
            zero_acc()
            plsc.subcore_barrier()

            if with_cnt:
                @pl.when(c == p // (_NPASS // 2))
                def _():
                    pltpu.sync_copy(ones_h, rows_v)

                    def cnt_body(j, carry2):
                        load_idx2(j)
                        pltpu.sync_copy(rows_v, acc_sh.at[idx2.at[0]],
                                        add=True)
                        return carry2

                    lax.fori_loop(0, nch, cnt_body, 0)
                    plsc.subcore_barrier()
                    pltpu.sync_copy(acc_sh.at[pl.ds(w0, wps)],
                                    cnt_h.at[pl.ds(p * _SEG + w0, wps)])
                    zero_acc()
                    plsc.subcore_barrier()
            return carry

        lax.fori_loop(0, _NPASS, pass_body, 0)

    return agg(table, src_all, dst3, zeros_acc,
               *((ones_rows,) if with_cnt else ()))


def kernel(x, edge_index, W_proj, b_proj, W0, b0, g0, be0, W1, b1, g1, be1):
    B, N, C = x.shape
    E = edge_index.shape[1]

    h_sc, t0 = _tc_proj(x, W_proj.T, b_proj, W0.T, b0)

    src = edge_index[0].astype(jnp.int32)
    dst = edge_index[1].astype(jnp.int32)
    ns = 16
    cps = E // _K // ns
    src_all = jnp.stack([src, src + _NPAD]).reshape(2, ns, cps, _K)
    dst3 = dst.reshape(ns, cps, _K)
    zeros_acc = jnp.zeros((_ACC, C), _F32)
    ones_rows = jnp.ones((_K, C), _F32)

    def flat(t):
        return t.reshape(2 * _NPAD, C)

    def unflat(t):
        return t.reshape(2, _NPAD, C)

    s0, cnt = _sc_agg(flat(t0), src_all, dst3, zeros_acc, ones_rows)
    t1 = _tc_post(unflat(s0), cnt, g0, be0, N, WT=W1.T, b=b1)
    s1 = _sc_agg(flat(t1), src_all, dst3, zeros_acc)
    out = _tc_post(unflat(s1), cnt, g1, be1, N, shortcut=h_sc)
    return out
